# baseline (speedup 1.0000x reference)
import functools

import jax
import jax.numpy as jnp
from jax import lax
from jax.experimental import pallas as pl
from jax.experimental.pallas import tpu as pltpu
from jax.experimental.pallas import tpu_sc as plsc

B = 16384
EMB = 64
NC = 2
NS = 16
NW = NC * NS
BPW = B // NW
CHUNK = 128
NCHUNK = BPW // CHUNK

TC_BLK = 2048


def _sc_gather(users, items, gmf_user_W, gmf_item_W, mlp_user_W, mlp_item_W):
    mesh = plsc.VectorSubcoreMesh(core_axis_name="c", subcore_axis_name="s")
    row_t = jax.ShapeDtypeStruct((B, EMB), jnp.float32)

    @functools.partial(
        pl.kernel,
        out_type=[row_t, row_t, row_t, row_t],
        mesh=mesh,
        compiler_params=pltpu.CompilerParams(use_tc_tiling_on_sc=False),
        scratch_types=[
            pltpu.VMEM((BPW,), jnp.int32),
            pltpu.VMEM((BPW,), jnp.int32),
            pltpu.VMEM((CHUNK, EMB), jnp.float32),
        ],
    )
    def gather_kernel(u_hbm, i_hbm, gu_hbm, gi_hbm, mu_hbm, mi_hbm,
                      o_gu, o_gi, o_mu, o_mi, idx_u, idx_i, rows):
        wid = lax.axis_index("s") * NC + lax.axis_index("c")
        base = wid * BPW
        pltpu.sync_copy(u_hbm.at[pl.ds(base, BPW)], idx_u)
        pltpu.sync_copy(i_hbm.at[pl.ds(base, BPW)], idx_i)
        for table, idx, out in ((gu_hbm, idx_u, o_gu),
                                (gi_hbm, idx_i, o_gi),
                                (mu_hbm, idx_u, o_mu),
                                (mi_hbm, idx_i, o_mi)):
            for c in range(NCHUNK):
                pltpu.sync_copy(table.at[idx.at[pl.ds(c * CHUNK, CHUNK)]], rows)
                pltpu.sync_copy(rows, out.at[pl.ds(base + c * CHUNK, CHUNK)])

    return gather_kernel(users, items, gmf_user_W, gmf_item_W,
                         mlp_user_W, mlp_item_W)


def _tc_body(gu_ref, gi_ref, mu_ref, mi_ref, w1u_ref, w1i_ref, b1_ref,
             w2_ref, b2_ref, w3_ref, b3_ref, o_ref):
    mu = mu_ref[...]
    mi = mi_ref[...]
    h1 = jnp.dot(mu, w1u_ref[...], preferred_element_type=jnp.float32)
    h1 = h1 + jnp.dot(mi, w1i_ref[...], preferred_element_type=jnp.float32)
    h1 = jnp.maximum(h1 + b1_ref[...], 0.0)
    h2 = jnp.dot(h1, w2_ref[...], preferred_element_type=jnp.float32)
    h2 = jnp.maximum(h2 + b2_ref[...], 0.0)
    mlp = jnp.dot(h2, w3_ref[...], preferred_element_type=jnp.float32)
    gmf = jnp.sum(gu_ref[...] * gi_ref[...], axis=1, keepdims=True)
    o_ref[...] = jax.nn.sigmoid(gmf + mlp + b3_ref[...])


def _tc_dense(gu, gi, mu, mi, W1, b1, W2, b2, W3, b3):
    w1u = W1[:, :EMB].T
    w1i = W1[:, EMB:].T
    w2t = W2.T
    w3t = W3.T
    b1r = b1.reshape(1, -1)
    b2r = b2.reshape(1, -1)
    b3r = b3.reshape(1, 1)

    grid = B // TC_BLK
    blk = lambda: pl.BlockSpec((TC_BLK, EMB), lambda i: (i, 0))
    full = lambda a: pl.BlockSpec(a.shape, lambda i: (0,) * a.ndim)
    out = pl.pallas_call(
        _tc_body,
        grid=(grid,),
        in_specs=[blk(), blk(), blk(), blk(),
                  full(w1u), full(w1i), full(b1r),
                  full(w2t), full(b2r), full(w3t), full(b3r)],
        out_specs=pl.BlockSpec((TC_BLK, 1), lambda i: (i, 0)),
        out_shape=jax.ShapeDtypeStruct((B, 1), jnp.float32),
    )(gu, gi, mu, mi, w1u, w1i, b1r, w2t, b2r, w3t, b3r)
    return out.reshape(B)


def kernel(users, items, gmf_user_W, gmf_item_W, mlp_user_W, mlp_item_W,
           W1, b1, W2, b2, W3, b3):
    users = users.astype(jnp.int32)
    items = items.astype(jnp.int32)
    gu, gi, mu, mi = _sc_gather(users, items, gmf_user_W, gmf_item_W,
                                mlp_user_W, mlp_item_W)
    return _tc_dense(gu, gi, mu, mi, W1, b1, W2, b2, W3, b3)


# speedup vs baseline: 1.4457x; 1.4457x over previous
import functools

import jax
import jax.numpy as jnp
from jax import lax
from jax.experimental import pallas as pl
from jax.experimental.pallas import tpu as pltpu
from jax.experimental.pallas import tpu_sc as plsc

B = 16384
EMB = 64
NC = 2
NS = 16
NW = NC * NS
BPW = B // NW
CHUNK = 128
NCHUNK = BPW // CHUNK

TC_BLK = 2048


def _sc_gather(users, items, gmf_user_W, gmf_item_W, mlp_user_W, mlp_item_W):
    mesh = plsc.VectorSubcoreMesh(core_axis_name="c", subcore_axis_name="s")
    row_t = jax.ShapeDtypeStruct((B, EMB), jnp.float32)
    buf_t = pltpu.VMEM((CHUNK, EMB), jnp.float32)

    @functools.partial(
        pl.kernel,
        out_type=[row_t, row_t, row_t, row_t],
        mesh=mesh,
        scratch_types=[
            pltpu.VMEM((BPW,), jnp.int32),
            pltpu.VMEM((BPW,), jnp.int32),
            buf_t, buf_t, buf_t, buf_t,
            pltpu.SemaphoreType.DMA,
        ],
    )
    def gather_kernel(u_hbm, i_hbm, gu_hbm, gi_hbm, mu_hbm, mi_hbm,
                      o_gu, o_gi, o_mu, o_mi,
                      idx_u, idx_i, b_gu, b_gi, b_mu, b_mi, sem):
        wid = lax.axis_index("s") * NC + lax.axis_index("c")
        base = wid * BPW
        pltpu.sync_copy(u_hbm.at[pl.ds(base, BPW)], idx_u)
        pltpu.sync_copy(i_hbm.at[pl.ds(base, BPW)], idx_i)

        for c in range(NCHUNK):
            off = c * CHUNK

            for idx, table, buf in ((idx_u, gu_hbm, b_gu),
                                    (idx_u, mu_hbm, b_mu),
                                    (idx_i, gi_hbm, b_gi),
                                    (idx_i, mi_hbm, b_mi)):
                @pl.loop(0, CHUNK, step=16)
                def _(r, idx=idx, table=table, buf=buf):
                    vec = idx[pl.ds(off + r, 16)]
                    for j in range(16):
                        pltpu.async_copy(table.at[pl.ds(vec[j], 1)],
                                         buf.at[pl.ds(r + j, 1)], sem)

            for buf in (b_gu, b_mu, b_gi, b_mi):
                pltpu.make_async_copy(gu_hbm.at[pl.ds(0, CHUNK)], buf, sem).wait()

            dsl = pl.ds(base + off, CHUNK)
            pltpu.sync_copy(b_gu, o_gu.at[dsl])
            pltpu.sync_copy(b_gi, o_gi.at[dsl])
            pltpu.sync_copy(b_mu, o_mu.at[dsl])
            pltpu.sync_copy(b_mi, o_mi.at[dsl])

    return gather_kernel(users, items, gmf_user_W, gmf_item_W,
                         mlp_user_W, mlp_item_W)


def _tc_body(gu_ref, gi_ref, mu_ref, mi_ref, w1u_ref, w1i_ref, b1_ref,
             w2_ref, b2_ref, w3_ref, b3_ref, o_ref):
    mu = mu_ref[...]
    mi = mi_ref[...]
    h1 = jnp.dot(mu, w1u_ref[...], preferred_element_type=jnp.float32)
    h1 = h1 + jnp.dot(mi, w1i_ref[...], preferred_element_type=jnp.float32)
    h1 = jnp.maximum(h1 + b1_ref[...], 0.0)
    h2 = jnp.dot(h1, w2_ref[...], preferred_element_type=jnp.float32)
    h2 = jnp.maximum(h2 + b2_ref[...], 0.0)
    mlp = jnp.dot(h2, w3_ref[...], preferred_element_type=jnp.float32)
    gmf = jnp.sum(gu_ref[...] * gi_ref[...], axis=1, keepdims=True)
    o_ref[...] = jax.nn.sigmoid(gmf + mlp + b3_ref[...])


def _tc_dense(gu, gi, mu, mi, W1, b1, W2, b2, W3, b3):
    w1u = W1[:, :EMB].T
    w1i = W1[:, EMB:].T
    w2t = W2.T
    w3t = W3.T
    b1r = b1.reshape(1, -1)
    b2r = b2.reshape(1, -1)
    b3r = b3.reshape(1, 1)

    grid = B // TC_BLK
    blk = lambda: pl.BlockSpec((TC_BLK, EMB), lambda i: (i, 0))
    full = lambda a: pl.BlockSpec(a.shape, lambda i: (0,) * a.ndim)
    out = pl.pallas_call(
        _tc_body,
        grid=(grid,),
        in_specs=[blk(), blk(), blk(), blk(),
                  full(w1u), full(w1i), full(b1r),
                  full(w2t), full(b2r), full(w3t), full(b3r)],
        out_specs=pl.BlockSpec((TC_BLK, 1), lambda i: (i, 0)),
        out_shape=jax.ShapeDtypeStruct((B, 1), jnp.float32),
    )(gu, gi, mu, mi, w1u, w1i, b1r, w2t, b2r, w3t, b3r)
    return out.reshape(B)


def kernel(users, items, gmf_user_W, gmf_item_W, mlp_user_W, mlp_item_W,
           W1, b1, W2, b2, W3, b3):
    users = users.astype(jnp.int32)
    items = items.astype(jnp.int32)
    gu, gi, mu, mi = _sc_gather(users, items, gmf_user_W, gmf_item_W,
                                mlp_user_W, mlp_item_W)
    return _tc_dense(gu, gi, mu, mi, W1, b1, W2, b2, W3, b3)


# speedup vs baseline: 1.7665x; 1.2220x over previous
import functools

import jax
import jax.numpy as jnp
import numpy as np
from jax import lax
from jax.experimental import pallas as pl
from jax.experimental.pallas import tpu as pltpu
from jax.experimental.pallas import tpu_sc as plsc

B = 16384
EMB = 64
NC = 2
NS = 16
NW = NC * NS
BPW = B // NW
CHUNK = 128
NCHUNK = BPW // CHUNK

N_ROWS = 100000
PREP_BLK = 2048
TC_BLK = 2048

HI_MASK = np.uint32(0xFFFF0000)


def _prep_body(a_ref, b_ref, o_ref):
    at = a_ref[...].T
    bt = b_ref[...].T
    hi = lax.bitcast_convert_type(at, jnp.uint32) & HI_MASK
    lo = lax.bitcast_convert_type(bt, jnp.uint32) >> 16
    o_ref[...] = lax.bitcast_convert_type(hi | lo, jnp.float32)


def _tc_prep(tT_a, tT_b):
    grid = (N_ROWS + PREP_BLK - 1) // PREP_BLK
    return pl.pallas_call(
        _prep_body,
        grid=(grid,),
        in_specs=[pl.BlockSpec((EMB, PREP_BLK), lambda i: (0, i)),
                  pl.BlockSpec((EMB, PREP_BLK), lambda i: (0, i))],
        out_specs=pl.BlockSpec((PREP_BLK, EMB), lambda i: (i, 0)),
        out_shape=jax.ShapeDtypeStruct((N_ROWS, EMB), jnp.float32),
    )(tT_a, tT_b)


def _sc_gather(users, items, U, V):
    mesh = plsc.VectorSubcoreMesh(core_axis_name="c", subcore_axis_name="s")
    row_t = jax.ShapeDtypeStruct((B, EMB), jnp.float32)
    buf_t = pltpu.VMEM((CHUNK, EMB), jnp.float32)

    @functools.partial(
        pl.kernel,
        out_type=[row_t, row_t],
        mesh=mesh,
        scratch_types=[
            pltpu.VMEM((BPW,), jnp.int32),
            pltpu.VMEM((BPW,), jnp.int32),
            buf_t, buf_t,
            pltpu.SemaphoreType.DMA,
        ],
    )
    def gather_kernel(u_hbm, i_hbm, U_hbm, V_hbm, o_u, o_v,
                      idx_u, idx_i, b_u, b_v, sem):
        wid = lax.axis_index("s") * NC + lax.axis_index("c")
        base = wid * BPW
        pltpu.sync_copy(u_hbm.at[pl.ds(base, BPW)], idx_u)
        pltpu.sync_copy(i_hbm.at[pl.ds(base, BPW)], idx_i)

        for c in range(NCHUNK):
            off = c * CHUNK

            for idx, table, buf in ((idx_u, U_hbm, b_u),
                                    (idx_i, V_hbm, b_v)):
                @pl.loop(0, CHUNK, step=16)
                def _(r, idx=idx, table=table, buf=buf):
                    vec = idx[pl.ds(off + r, 16)]
                    for j in range(16):
                        pltpu.async_copy(table.at[pl.ds(vec[j], 1)],
                                         buf.at[pl.ds(r + j, 1)], sem)

            for buf in (b_u, b_v):
                pltpu.make_async_copy(U_hbm.at[pl.ds(0, CHUNK)], buf, sem).wait()

            dsl = pl.ds(base + off, CHUNK)
            pltpu.sync_copy(b_u, o_u.at[dsl])
            pltpu.sync_copy(b_v, o_v.at[dsl])

    return gather_kernel(users, items, U, V)


def _unpack(x):
    w = lax.bitcast_convert_type(x, jnp.uint32)
    hi = lax.bitcast_convert_type(w & HI_MASK, jnp.float32)
    lo = lax.bitcast_convert_type(w << 16, jnp.float32)
    return hi, lo


def _tc_body(u_ref, v_ref, w1u_ref, w1i_ref, b1_ref, w2_ref, b2_ref,
             w3_ref, b3_ref, o_ref):
    gmf_u, mlp_u = _unpack(u_ref[...])
    gmf_v, mlp_v = _unpack(v_ref[...])
    h1 = jnp.dot(mlp_u, w1u_ref[...], preferred_element_type=jnp.float32)
    h1 = h1 + jnp.dot(mlp_v, w1i_ref[...], preferred_element_type=jnp.float32)
    h1 = jnp.maximum(h1 + b1_ref[...], 0.0)
    h2 = jnp.dot(h1, w2_ref[...], preferred_element_type=jnp.float32)
    h2 = jnp.maximum(h2 + b2_ref[...], 0.0)
    mlp = jnp.dot(h2, w3_ref[...], preferred_element_type=jnp.float32)
    gmf = jnp.sum(gmf_u * gmf_v, axis=1, keepdims=True)
    o_ref[...] = jax.nn.sigmoid(gmf + mlp + b3_ref[...])


def _tc_dense(gu, gi, W1, b1, W2, b2, W3, b3):
    w1u = W1[:, :EMB].T
    w1i = W1[:, EMB:].T
    w2t = W2.T
    w3t = W3.T
    b1r = b1.reshape(1, -1)
    b2r = b2.reshape(1, -1)
    b3r = b3.reshape(1, 1)

    grid = B // TC_BLK
    blk = lambda: pl.BlockSpec((TC_BLK, EMB), lambda i: (i, 0))
    full = lambda a: pl.BlockSpec(a.shape, lambda i: (0,) * a.ndim)
    out = pl.pallas_call(
        _tc_body,
        grid=(grid,),
        in_specs=[blk(), blk(),
                  full(w1u), full(w1i), full(b1r),
                  full(w2t), full(b2r), full(w3t), full(b3r)],
        out_specs=pl.BlockSpec((TC_BLK, 1), lambda i: (i, 0)),
        out_shape=jax.ShapeDtypeStruct((B, 1), jnp.float32),
    )(gu, gi, w1u, w1i, b1r, w2t, b2r, w3t, b3r)
    return out.reshape(B)


def kernel(users, items, gmf_user_W, gmf_item_W, mlp_user_W, mlp_item_W,
           W1, b1, W2, b2, W3, b3):
    users = users.astype(jnp.int32)
    items = items.astype(jnp.int32)
    U = _tc_prep(gmf_user_W.T, mlp_user_W.T)
    V = _tc_prep(gmf_item_W.T, mlp_item_W.T)
    gu, gi = _sc_gather(users, items, U, V)
    return _tc_dense(gu, gi, W1, b1, W2, b2, W3, b3)


# speedup vs baseline: 2.2632x; 1.2811x over previous
import functools

import jax
import jax.numpy as jnp
import numpy as np
from jax import lax
from jax.experimental import pallas as pl
from jax.experimental.pallas import tpu as pltpu
from jax.experimental.pallas import tpu_sc as plsc

B = 16384
EMB = 64
NC = 2
NS = 16
NW = NC * NS
BPW = B // NW
CHUNK = 128
NCHUNK = BPW // CHUNK

N_ROWS = 100000
PREP_BLK = 2048
TC_BLK = 2048

HI_MASK = np.uint32(0xFFFF0000)


def _transpose_mxu(x_ref, eye):
    return lax.dot_general(x_ref[...].astype(jnp.bfloat16), eye,
                           (((0,), (0,)), ((), ())),
                           preferred_element_type=jnp.float32)


def _prep_body(ga_ref, ma_ref, gb_ref, mb_ref, eye_ref, u_ref, v_ref):
    eye = eye_ref[...]
    for g_ref, m_ref, o_ref in ((ga_ref, ma_ref, u_ref),
                                (gb_ref, mb_ref, v_ref)):
        gt = _transpose_mxu(g_ref, eye)
        mt = _transpose_mxu(m_ref, eye)
        hi = lax.bitcast_convert_type(gt, jnp.uint32)
        lo = lax.bitcast_convert_type(mt, jnp.uint32) >> 16
        o_ref[...] = lax.bitcast_convert_type(hi | lo, jnp.float32)


def _tc_prep(guT, muT, giT, miT):
    grid = (N_ROWS + PREP_BLK - 1) // PREP_BLK
    eye = jnp.eye(EMB, dtype=jnp.bfloat16)
    tin = lambda: pl.BlockSpec((EMB, PREP_BLK), lambda i: (0, i))
    tout = lambda: pl.BlockSpec((PREP_BLK, EMB), lambda i: (i, 0))
    out_t = jax.ShapeDtypeStruct((N_ROWS, EMB), jnp.float32)
    return pl.pallas_call(
        _prep_body,
        grid=(grid,),
        in_specs=[tin(), tin(), tin(), tin(),
                  pl.BlockSpec((EMB, EMB), lambda i: (0, 0))],
        out_specs=[tout(), tout()],
        out_shape=[out_t, out_t],
    )(guT, muT, giT, miT, eye)


def _sc_gather(users, items, U, V):
    mesh = plsc.VectorSubcoreMesh(core_axis_name="c", subcore_axis_name="s")
    row_t = jax.ShapeDtypeStruct((B, EMB), jnp.float32)
    buf_t = pltpu.VMEM((CHUNK, EMB), jnp.float32)

    @functools.partial(
        pl.kernel,
        out_type=[row_t, row_t],
        mesh=mesh,
        scratch_types=[
            pltpu.VMEM((BPW,), jnp.int32),
            pltpu.VMEM((BPW,), jnp.int32),
            buf_t, buf_t,
            pltpu.SemaphoreType.DMA,
        ],
    )
    def gather_kernel(u_hbm, i_hbm, U_hbm, V_hbm, o_u, o_v,
                      idx_u, idx_i, b_u, b_v, sem):
        wid = lax.axis_index("s") * NC + lax.axis_index("c")
        base = wid * BPW
        pltpu.sync_copy(u_hbm.at[pl.ds(base, BPW)], idx_u)
        pltpu.sync_copy(i_hbm.at[pl.ds(base, BPW)], idx_i)

        for c in range(NCHUNK):
            off = c * CHUNK

            for idx, table, buf in ((idx_u, U_hbm, b_u),
                                    (idx_i, V_hbm, b_v)):
                @pl.loop(0, CHUNK, step=16)
                def _(r, idx=idx, table=table, buf=buf):
                    vec = idx[pl.ds(off + r, 16)]
                    for j in range(16):
                        pltpu.async_copy(table.at[pl.ds(vec[j], 1)],
                                         buf.at[pl.ds(r + j, 1)], sem)

            for buf in (b_u, b_v):
                pltpu.make_async_copy(U_hbm.at[pl.ds(0, CHUNK)], buf, sem).wait()

            dsl = pl.ds(base + off, CHUNK)
            pltpu.sync_copy(b_u, o_u.at[dsl])
            pltpu.sync_copy(b_v, o_v.at[dsl])

    return gather_kernel(users, items, U, V)


def _unpack(x):
    w = lax.bitcast_convert_type(x, jnp.uint32)
    hi = lax.bitcast_convert_type(w & HI_MASK, jnp.float32)
    lo = lax.bitcast_convert_type(w << 16, jnp.float32)
    return hi, lo


def _tc_body(u_ref, v_ref, w1u_ref, w1i_ref, b1_ref, w2_ref, b2_ref,
             w3_ref, b3_ref, o_ref):
    gmf_u, mlp_u = _unpack(u_ref[...])
    gmf_v, mlp_v = _unpack(v_ref[...])
    h1 = jnp.dot(mlp_u, w1u_ref[...], preferred_element_type=jnp.float32)
    h1 = h1 + jnp.dot(mlp_v, w1i_ref[...], preferred_element_type=jnp.float32)
    h1 = jnp.maximum(h1 + b1_ref[...], 0.0)
    h2 = jnp.dot(h1, w2_ref[...], preferred_element_type=jnp.float32)
    h2 = jnp.maximum(h2 + b2_ref[...], 0.0)
    mlp = jnp.dot(h2, w3_ref[...], preferred_element_type=jnp.float32)
    gmf = jnp.sum(gmf_u * gmf_v, axis=1, keepdims=True)
    o_ref[...] = jax.nn.sigmoid(gmf + mlp + b3_ref[...])


def _tc_dense(gu, gi, W1, b1, W2, b2, W3, b3):
    w1u = W1[:, :EMB].T
    w1i = W1[:, EMB:].T
    w2t = W2.T
    w3t = W3.T
    b1r = b1.reshape(1, -1)
    b2r = b2.reshape(1, -1)
    b3r = b3.reshape(1, 1)

    grid = B // TC_BLK
    blk = lambda: pl.BlockSpec((TC_BLK, EMB), lambda i: (i, 0))
    full = lambda a: pl.BlockSpec(a.shape, lambda i: (0,) * a.ndim)
    out = pl.pallas_call(
        _tc_body,
        grid=(grid,),
        in_specs=[blk(), blk(),
                  full(w1u), full(w1i), full(b1r),
                  full(w2t), full(b2r), full(w3t), full(b3r)],
        out_specs=pl.BlockSpec((TC_BLK, 1), lambda i: (i, 0)),
        out_shape=jax.ShapeDtypeStruct((B, 1), jnp.float32),
    )(gu, gi, w1u, w1i, b1r, w2t, b2r, w3t, b3r)
    return out.reshape(B)


def kernel(users, items, gmf_user_W, gmf_item_W, mlp_user_W, mlp_item_W,
           W1, b1, W2, b2, W3, b3):
    users = users.astype(jnp.int32)
    items = items.astype(jnp.int32)
    U, V = _tc_prep(gmf_user_W.T, mlp_user_W.T, gmf_item_W.T, mlp_item_W.T)
    gu, gi = _sc_gather(users, items, U, V)
    return _tc_dense(gu, gi, W1, b1, W2, b2, W3, b3)


# speedup vs baseline: 2.6250x; 1.1599x over previous
import functools

import jax
import jax.numpy as jnp
import numpy as np
from jax import lax
from jax.experimental import pallas as pl
from jax.experimental.pallas import tpu as pltpu
from jax.experimental.pallas import tpu_sc as plsc

B = 16384
EMB = 64
NC = 2
NS = 16
NW = NC * NS
BPW = B // NW
CHUNK = 128
NCHUNK = BPW // CHUNK

N_ROWS = 100000
PREP_BLK = 8192
TC_BLK = 2048

HI_MASK = np.uint32(0xFFFF0000)


def _transpose_mxu(x_ref, eye):
    return lax.dot_general(x_ref[...].astype(jnp.bfloat16), eye,
                           (((0,), (0,)), ((), ())),
                           preferred_element_type=jnp.float32)


def _prep_body(ga_ref, ma_ref, gb_ref, mb_ref, eye_ref, u_ref, v_ref):
    eye = eye_ref[...]
    for g_ref, m_ref, o_ref in ((ga_ref, ma_ref, u_ref),
                                (gb_ref, mb_ref, v_ref)):
        gt = _transpose_mxu(g_ref, eye)
        mt = _transpose_mxu(m_ref, eye)
        hi = lax.bitcast_convert_type(gt, jnp.uint32)
        lo = lax.bitcast_convert_type(mt, jnp.uint32) >> 16
        o_ref[...] = lax.bitcast_convert_type(hi | lo, jnp.float32)


def _tc_prep(guT, muT, giT, miT):
    grid = (N_ROWS + PREP_BLK - 1) // PREP_BLK
    eye = jnp.eye(EMB, dtype=jnp.bfloat16)
    tin = lambda: pl.BlockSpec((EMB, PREP_BLK), lambda i: (0, i))
    tout = lambda: pl.BlockSpec((PREP_BLK, EMB), lambda i: (i, 0))
    out_t = jax.ShapeDtypeStruct((N_ROWS, EMB), jnp.float32)
    return pl.pallas_call(
        _prep_body,
        grid=(grid,),
        in_specs=[tin(), tin(), tin(), tin(),
                  pl.BlockSpec((EMB, EMB), lambda i: (0, 0))],
        out_specs=[tout(), tout()],
        out_shape=[out_t, out_t],
    )(guT, muT, giT, miT, eye)


def _sc_gather(users, items, U, V):
    mesh = plsc.VectorSubcoreMesh(core_axis_name="c", subcore_axis_name="s")
    row_t = jax.ShapeDtypeStruct((B, EMB), jnp.float32)
    buf_t = pltpu.VMEM((CHUNK, EMB), jnp.float32)

    @functools.partial(
        pl.kernel,
        out_type=[row_t, row_t],
        mesh=mesh,
        scratch_types=[
            pltpu.VMEM((BPW,), jnp.int32),
            pltpu.VMEM((BPW,), jnp.int32),
            buf_t, buf_t,
            pltpu.SemaphoreType.DMA,
        ],
    )
    def gather_kernel(u_hbm, i_hbm, U_hbm, V_hbm, o_u, o_v,
                      idx_u, idx_i, b_u, b_v, sem):
        wid = lax.axis_index("s") * NC + lax.axis_index("c")
        base = wid * BPW
        pltpu.sync_copy(u_hbm.at[pl.ds(base, BPW)], idx_u)
        pltpu.sync_copy(i_hbm.at[pl.ds(base, BPW)], idx_i)

        for c in range(NCHUNK):
            off = c * CHUNK

            for idx, table, buf in ((idx_u, U_hbm, b_u),
                                    (idx_i, V_hbm, b_v)):
                @pl.loop(0, CHUNK, step=16)
                def _(r, idx=idx, table=table, buf=buf):
                    vec = idx[pl.ds(off + r, 16)]
                    for j in range(16):
                        pltpu.async_copy(table.at[pl.ds(vec[j], 1)],
                                         buf.at[pl.ds(r + j, 1)], sem)

            for buf in (b_u, b_v):
                pltpu.make_async_copy(U_hbm.at[pl.ds(0, CHUNK)], buf, sem).wait()

            dsl = pl.ds(base + off, CHUNK)
            pltpu.sync_copy(b_u, o_u.at[dsl])
            pltpu.sync_copy(b_v, o_v.at[dsl])

    return gather_kernel(users, items, U, V)


def _unpack(x):
    w = lax.bitcast_convert_type(x, jnp.uint32)
    hi = lax.bitcast_convert_type(w & HI_MASK, jnp.float32)
    lo = lax.bitcast_convert_type(w << 16, jnp.float32)
    return hi, lo


def _tc_body(u_ref, v_ref, w1u_ref, w1i_ref, b1_ref, w2_ref, b2_ref,
             w3_ref, b3_ref, o_ref):
    gmf_u, mlp_u = _unpack(u_ref[...])
    gmf_v, mlp_v = _unpack(v_ref[...])
    h1 = jnp.dot(mlp_u, w1u_ref[...], preferred_element_type=jnp.float32)
    h1 = h1 + jnp.dot(mlp_v, w1i_ref[...], preferred_element_type=jnp.float32)
    h1 = jnp.maximum(h1 + b1_ref[...], 0.0)
    h2 = jnp.dot(h1, w2_ref[...], preferred_element_type=jnp.float32)
    h2 = jnp.maximum(h2 + b2_ref[...], 0.0)
    mlp = jnp.dot(h2, w3_ref[...], preferred_element_type=jnp.float32)
    gmf = jnp.sum(gmf_u * gmf_v, axis=1, keepdims=True)
    o_ref[...] = jax.nn.sigmoid(gmf + mlp + b3_ref[...])[:, 0]


def _tc_dense(gu, gi, W1, b1, W2, b2, W3, b3):
    w1u = W1[:, :EMB].T
    w1i = W1[:, EMB:].T
    w2t = W2.T
    w3t = W3.T
    b1r = b1.reshape(1, -1)
    b2r = b2.reshape(1, -1)
    b3r = b3.reshape(1, 1)

    grid = B // TC_BLK
    blk = lambda: pl.BlockSpec((TC_BLK, EMB), lambda i: (i, 0))
    full = lambda a: pl.BlockSpec(a.shape, lambda i: (0,) * a.ndim)
    out = pl.pallas_call(
        _tc_body,
        grid=(grid,),
        in_specs=[blk(), blk(),
                  full(w1u), full(w1i), full(b1r),
                  full(w2t), full(b2r), full(w3t), full(b3r)],
        out_specs=pl.BlockSpec((TC_BLK,), lambda i: (i,)),
        out_shape=jax.ShapeDtypeStruct((B,), jnp.float32),
    )(gu, gi, w1u, w1i, b1r, w2t, b2r, w3t, b3r)
    return out


def kernel(users, items, gmf_user_W, gmf_item_W, mlp_user_W, mlp_item_W,
           W1, b1, W2, b2, W3, b3):
    users = users.astype(jnp.int32)
    items = items.astype(jnp.int32)
    U, V = _tc_prep(gmf_user_W.T, mlp_user_W.T, gmf_item_W.T, mlp_item_W.T)
    gu, gi = _sc_gather(users, items, U, V)
    return _tc_dense(gu, gi, W1, b1, W2, b2, W3, b3)


# speedup vs baseline: 2.6514x; 1.0101x over previous
import functools

import jax
import jax.numpy as jnp
import numpy as np
from jax import lax
from jax.experimental import pallas as pl
from jax.experimental.pallas import tpu as pltpu
from jax.experimental.pallas import tpu_sc as plsc

B = 16384
EMB = 64
NC = 2
NS = 16
NW = NC * NS
BPW = B // NW
CHUNK = 128
NCHUNK = BPW // CHUNK

N_ROWS = 100000
PREP_BLK = 16384
TC_BLK = 4096

HI_MASK = np.uint32(0xFFFF0000)


def _transpose_mxu(x_ref, eye):
    return lax.dot_general(x_ref[...].astype(jnp.bfloat16), eye,
                           (((0,), (0,)), ((), ())),
                           preferred_element_type=jnp.float32)


def _prep_body(g_ref, m_ref, eye_ref, o_ref):
    eye = eye_ref[...]
    gt = _transpose_mxu(g_ref, eye)
    mt = _transpose_mxu(m_ref, eye)
    hi = lax.bitcast_convert_type(gt, jnp.uint32)
    lo = lax.bitcast_convert_type(mt, jnp.uint32) >> 16
    o_ref[...] = lax.bitcast_convert_type(hi | lo, jnp.float32)


def _tc_prep(gT, mT):
    grid = (N_ROWS + PREP_BLK - 1) // PREP_BLK
    eye = jnp.eye(EMB, dtype=jnp.bfloat16)
    return pl.pallas_call(
        _prep_body,
        grid=(grid,),
        in_specs=[pl.BlockSpec((EMB, PREP_BLK), lambda i: (0, i)),
                  pl.BlockSpec((EMB, PREP_BLK), lambda i: (0, i)),
                  pl.BlockSpec((EMB, EMB), lambda i: (0, 0))],
        out_specs=pl.BlockSpec((PREP_BLK, EMB), lambda i: (i, 0)),
        out_shape=jax.ShapeDtypeStruct((N_ROWS, EMB), jnp.float32),
    )(gT, mT, eye)


def _sc_gather(idxs, T):
    mesh = plsc.VectorSubcoreMesh(core_axis_name="c", subcore_axis_name="s")
    row_t = jax.ShapeDtypeStruct((B, EMB), jnp.float32)
    buf_t = pltpu.VMEM((CHUNK, EMB), jnp.float32)

    @functools.partial(
        pl.kernel,
        out_type=row_t,
        mesh=mesh,
        scratch_types=[
            pltpu.VMEM((BPW,), jnp.int32),
            buf_t, buf_t,
            pltpu.SemaphoreType.DMA,
            pltpu.SemaphoreType.DMA,
        ],
    )
    def gather_kernel(i_hbm, T_hbm, o_hbm, idx, b0, b1, s0, s1):
        wid = lax.axis_index("s") * NC + lax.axis_index("c")
        base = wid * BPW
        pltpu.sync_copy(i_hbm.at[pl.ds(base, BPW)], idx)

        def fire(c, buf, sem):
            off = c * CHUNK

            @pl.loop(0, CHUNK, step=16)
            def _(r):
                vec = idx[pl.ds(off + r, 16)]
                for j in range(16):
                    pltpu.async_copy(T_hbm.at[pl.ds(vec[j], 1)],
                                     buf.at[pl.ds(r + j, 1)], sem)

        def drain_store(c, buf, sem):
            pltpu.make_async_copy(T_hbm.at[pl.ds(0, CHUNK)], buf, sem).wait()
            pltpu.sync_copy(buf, o_hbm.at[pl.ds(base + c * CHUNK, CHUNK)])

        fire(0, b0, s0)
        for c in range(NCHUNK):
            if c + 1 < NCHUNK:
                fire(c + 1, (b0, b1)[(c + 1) % 2], (s0, s1)[(c + 1) % 2])
            drain_store(c, (b0, b1)[c % 2], (s0, s1)[c % 2])

    return gather_kernel(idxs, T)


def _unpack(x):
    w = lax.bitcast_convert_type(x, jnp.uint32)
    hi = lax.bitcast_convert_type(w & HI_MASK, jnp.float32)
    lo = lax.bitcast_convert_type(w << 16, jnp.float32)
    return hi, lo


def _bf16(x):
    return x.astype(jnp.bfloat16)


def _tc_body(u_ref, v_ref, w1u_ref, w1i_ref, b1_ref, w2_ref, b2_ref,
             w3_ref, b3_ref, o_ref):
    gmf_u, mlp_u = _unpack(u_ref[...])
    gmf_v, mlp_v = _unpack(v_ref[...])
    h1 = jnp.dot(_bf16(mlp_u), w1u_ref[...], preferred_element_type=jnp.float32)
    h1 = h1 + jnp.dot(_bf16(mlp_v), w1i_ref[...],
                      preferred_element_type=jnp.float32)
    h1 = jnp.maximum(h1 + b1_ref[...], 0.0)
    h2 = jnp.dot(_bf16(h1), w2_ref[...], preferred_element_type=jnp.float32)
    h2 = jnp.maximum(h2 + b2_ref[...], 0.0)
    mlp = jnp.dot(_bf16(h2), w3_ref[...], preferred_element_type=jnp.float32)
    gmf = jnp.sum(gmf_u * gmf_v, axis=1, keepdims=True)
    o_ref[...] = jax.nn.sigmoid(gmf + mlp + b3_ref[...])[:, 0]


def _tc_dense(gu, gi, W1, b1, W2, b2, W3, b3):
    w1u = W1[:, :EMB].T.astype(jnp.bfloat16)
    w1i = W1[:, EMB:].T.astype(jnp.bfloat16)
    w2t = W2.T.astype(jnp.bfloat16)
    w3t = W3.T.astype(jnp.bfloat16)
    b1r = b1.reshape(1, -1)
    b2r = b2.reshape(1, -1)
    b3r = b3.reshape(1, 1)

    grid = B // TC_BLK
    blk = lambda: pl.BlockSpec((TC_BLK, EMB), lambda i: (i, 0))
    full = lambda a: pl.BlockSpec(a.shape, lambda i: (0,) * a.ndim)
    out = pl.pallas_call(
        _tc_body,
        grid=(grid,),
        in_specs=[blk(), blk(),
                  full(w1u), full(w1i), full(b1r),
                  full(w2t), full(b2r), full(w3t), full(b3r)],
        out_specs=pl.BlockSpec((TC_BLK,), lambda i: (i,)),
        out_shape=jax.ShapeDtypeStruct((B,), jnp.float32),
    )(gu, gi, w1u, w1i, b1r, w2t, b2r, w3t, b3r)
    return out


def kernel(users, items, gmf_user_W, gmf_item_W, mlp_user_W, mlp_item_W,
           W1, b1, W2, b2, W3, b3):
    users = users.astype(jnp.int32)
    items = items.astype(jnp.int32)
    U = _tc_prep(gmf_user_W.T, mlp_user_W.T)
    gu = _sc_gather(users, U)
    V = _tc_prep(gmf_item_W.T, mlp_item_W.T)
    gi = _sc_gather(items, V)
    return _tc_dense(gu, gi, W1, b1, W2, b2, W3, b3)


# speedup vs baseline: 2.8472x; 1.0738x over previous
import functools

import jax
import jax.numpy as jnp
import numpy as np
from jax import lax
from jax.experimental import pallas as pl
from jax.experimental.pallas import tpu as pltpu
from jax.experimental.pallas import tpu_sc as plsc

B = 16384
EMB = 64
NC = 2
NS = 16
NW = NC * NS
BPW = B // NW
CHUNK = 128
NCHUNK = BPW // CHUNK

N_ROWS = 100000
PREP_BLK = 16384
TC_BLK = 4096

HI_MASK = np.uint32(0xFFFF0000)


def _transpose_mxu(x_ref, eye):
    return lax.dot_general(x_ref[...].astype(jnp.bfloat16), eye,
                           (((0,), (0,)), ((), ())),
                           preferred_element_type=jnp.float32)


def _prep_body(g_ref, m_ref, eye_ref, o_ref):
    eye = eye_ref[...]
    gt = _transpose_mxu(g_ref, eye)
    mt = _transpose_mxu(m_ref, eye)
    hi = lax.bitcast_convert_type(gt, jnp.uint32)
    lo = lax.bitcast_convert_type(mt, jnp.uint32) >> 16
    o_ref[...] = lax.bitcast_convert_type(hi | lo, jnp.float32)


def _tc_prep(gT, mT):
    grid = (N_ROWS + PREP_BLK - 1) // PREP_BLK
    eye = jnp.eye(EMB, dtype=jnp.bfloat16)
    return pl.pallas_call(
        _prep_body,
        grid=(grid,),
        in_specs=[pl.BlockSpec((EMB, PREP_BLK), lambda i: (0, i)),
                  pl.BlockSpec((EMB, PREP_BLK), lambda i: (0, i)),
                  pl.BlockSpec((EMB, EMB), lambda i: (0, 0))],
        out_specs=pl.BlockSpec((PREP_BLK, EMB), lambda i: (i, 0)),
        out_shape=jax.ShapeDtypeStruct((N_ROWS, EMB), jnp.float32),
    )(gT, mT, eye)


def _sc_gather(idxs, T):
    mesh = plsc.VectorSubcoreMesh(core_axis_name="c", subcore_axis_name="s")
    row_t = jax.ShapeDtypeStruct((B, EMB), jnp.float32)
    buf_t = pltpu.VMEM((CHUNK, EMB), jnp.float32)

    @functools.partial(
        pl.kernel,
        out_type=row_t,
        mesh=mesh,
        scratch_types=[
            pltpu.VMEM((BPW,), jnp.int32),
            buf_t, buf_t,
            pltpu.SemaphoreType.DMA,
            pltpu.SemaphoreType.DMA,
        ],
    )
    def gather_kernel(i_hbm, T_hbm, o_hbm, idx, b0, b1, s0, s1):
        wid = lax.axis_index("s") * NC + lax.axis_index("c")
        base = wid * BPW
        pltpu.sync_copy(i_hbm.at[pl.ds(base, BPW)], idx)

        def fire(c, buf, sem):
            off = c * CHUNK

            @pl.loop(0, CHUNK, step=16)
            def _(r):
                vec = idx[pl.ds(off + r, 16)]
                for j in range(16):
                    pltpu.async_copy(T_hbm.at[pl.ds(vec[j], 1)],
                                     buf.at[pl.ds(r + j, 1)], sem)

        def drain_store(c, buf, sem):
            pltpu.make_async_copy(T_hbm.at[pl.ds(0, CHUNK)], buf, sem).wait()
            pltpu.sync_copy(buf, o_hbm.at[pl.ds(base + c * CHUNK, CHUNK)])

        fire(0, b0, s0)
        for c in range(NCHUNK):
            if c + 1 < NCHUNK:
                fire(c + 1, (b0, b1)[(c + 1) % 2], (s0, s1)[(c + 1) % 2])
            drain_store(c, (b0, b1)[c % 2], (s0, s1)[c % 2])

    return gather_kernel(idxs, T)


def _unpack(x):
    w = lax.bitcast_convert_type(x, jnp.uint32)
    hi = lax.bitcast_convert_type(w & HI_MASK, jnp.float32)
    lo = lax.bitcast_convert_type(w << 16, jnp.float32)
    return hi, lo


def _bf16(x):
    return x.astype(jnp.bfloat16)


def _dotT(w, x):
    return lax.dot_general(w, x, (((1,), (1,)), ((), ())),
                           preferred_element_type=jnp.float32)


def _tc_body(u_ref, v_ref, w1u_ref, w1i_ref, eye_ref, b1_ref, w2_ref,
             b2_ref, w3_ref, b3_ref, o_ref):
    gmf_u, mlp_u = _unpack(u_ref[...])
    gmf_v, mlp_v = _unpack(v_ref[...])
    h1 = _dotT(w1u_ref[...], _bf16(mlp_u)) + _dotT(w1i_ref[...], _bf16(mlp_v))
    h1 = jnp.maximum(h1 + b1_ref[...], 0.0)
    h2 = jnp.dot(w2_ref[...], _bf16(h1), preferred_element_type=jnp.float32)
    h2 = jnp.maximum(h2 + b2_ref[...], 0.0)
    mlp = jnp.dot(w3_ref[...], _bf16(h2),
                  preferred_element_type=jnp.float32)
    pT = _dotT(eye_ref[...], _bf16(gmf_u * gmf_v))
    gmf = jnp.sum(pT, axis=0, keepdims=True)
    o_ref[...] = jax.nn.sigmoid(gmf + mlp + b3_ref[...])[0]


def _tc_dense(gu, gi, W1, b1, W2, b2, W3, b3):
    w1u = W1[:, :EMB].astype(jnp.bfloat16)
    w1i = W1[:, EMB:].astype(jnp.bfloat16)
    w2b = W2.astype(jnp.bfloat16)
    w3b = W3.astype(jnp.bfloat16)
    eye = jnp.eye(EMB, dtype=jnp.bfloat16)
    b1c = b1.reshape(-1, 1)
    b2c = b2.reshape(-1, 1)
    b3c = b3.reshape(1, 1)

    grid = B // TC_BLK
    blk = lambda: pl.BlockSpec((TC_BLK, EMB), lambda i: (i, 0))
    full = lambda a: pl.BlockSpec(a.shape, lambda i: (0,) * a.ndim)
    out = pl.pallas_call(
        _tc_body,
        grid=(grid,),
        in_specs=[blk(), blk(),
                  full(w1u), full(w1i), full(eye), full(b1c),
                  full(w2b), full(b2c), full(w3b), full(b3c)],
        out_specs=pl.BlockSpec((TC_BLK,), lambda i: (i,)),
        out_shape=jax.ShapeDtypeStruct((B,), jnp.float32),
    )(gu, gi, w1u, w1i, eye, b1c, w2b, b2c, w3b, b3c)
    return out


def kernel(users, items, gmf_user_W, gmf_item_W, mlp_user_W, mlp_item_W,
           W1, b1, W2, b2, W3, b3):
    users = users.astype(jnp.int32)
    items = items.astype(jnp.int32)
    U = _tc_prep(gmf_user_W.T, mlp_user_W.T)
    gu = _sc_gather(users, U)
    V = _tc_prep(gmf_item_W.T, mlp_item_W.T)
    gi = _sc_gather(items, V)
    return _tc_dense(gu, gi, W1, b1, W2, b2, W3, b3)
